# baseline (speedup 1.0000x reference)
import functools

import jax
import jax.numpy as jnp
from jax import lax
from jax.experimental import pallas as pl
from jax.experimental.pallas import tpu as pltpu
from jax.experimental.pallas import tpu_sc as plsc

_F32 = jnp.float32
_CHUNK = 128
_DEGW = 128
_BLK = 1000


def _sc_dims():
    info = plsc.get_sparse_core_info()
    return info.num_cores, info.num_subcores


@functools.lru_cache(maxsize=None)
def _make_deg_kernel(n_pad, ept):
    nc, ns = _sc_dims()
    rpt = n_pad // ns
    nchunks = ept // _CHUNK
    mesh = plsc.VectorSubcoreMesh(core_axis_name="c", subcore_axis_name="s")

    npairs = nchunks // 2

    @functools.partial(
        pl.kernel,
        out_type=jax.ShapeDtypeStruct((nc, n_pad, _DEGW), _F32),
        mesh=mesh,
        scratch_types=[
            pltpu.VMEM_SHARED((n_pad, _DEGW), _F32),
            pltpu.VMEM((_CHUNK,), jnp.int32),
            pltpu.VMEM((_CHUNK,), jnp.int32),
            pltpu.VMEM((_CHUNK, _DEGW), _F32),
            pltpu.SemaphoreType.DMA,
            pltpu.SemaphoreType.DMA,
        ],
    )
    def deg_kernel(dst_hbm, zeros_hbm, ones_hbm, out_hbm,
                   acc_sh, dst0, dst1, ones_v, ssem0, ssem1):
        c = lax.axis_index("c")
        s = lax.axis_index("s")
        wid = s * nc + c
        base = wid * ept
        dst_v, ssem = (dst0, dst1), (ssem0, ssem1)
        pltpu.sync_copy(ones_hbm, ones_v)
        for b in range(2):
            pltpu.sync_copy(dst_hbm.at[pl.ds(base + b * _CHUNK, _CHUNK)],
                            dst_v[b])
        pltpu.sync_copy(zeros_hbm.at[pl.ds(s * rpt, rpt)],
                        acc_sh.at[pl.ds(s * rpt, rpt)])
        plsc.subcore_barrier()

        def body(i, carry):
            for b in range(2):
                off2 = base + (2 * i + b + 2) * _CHUNK
                pltpu.async_copy(ones_v, acc_sh.at[dst_v[b]], ssem[b],
                                 add=True)

                @pl.when(i < npairs - 1)
                def _(b=b, off2=off2):
                    pltpu.make_async_copy(ones_v, acc_sh.at[dst_v[b]],
                                          ssem[b]).wait()
                    pltpu.sync_copy(dst_hbm.at[pl.ds(off2, _CHUNK)], dst_v[b])
            return carry

        lax.fori_loop(0, npairs, body, 0)
        for b in range(2):
            pltpu.make_async_copy(ones_v, acc_sh.at[dst_v[b]],
                                  ssem[b]).wait()
        plsc.subcore_barrier()
        pltpu.sync_copy(acc_sh.at[pl.ds(s * rpt, rpt)],
                        out_hbm.at[c, pl.ds(s * rpt, rpt)])

    return deg_kernel


@functools.lru_cache(maxsize=None)
def _make_agg_kernel(n, n_pad, d, ept):
    nc, ns = _sc_dims()
    rpt = n_pad // ns
    nchunks = ept // _CHUNK
    npairs = nchunks // 2
    mesh = plsc.VectorSubcoreMesh(core_axis_name="c", subcore_axis_name="s")

    @functools.partial(
        pl.kernel,
        out_type=jax.ShapeDtypeStruct((nc, n_pad, d), _F32),
        mesh=mesh,
        scratch_types=[
            pltpu.VMEM_SHARED((n_pad, d), _F32),
            pltpu.VMEM((_CHUNK,), jnp.int32),
            pltpu.VMEM((_CHUNK,), jnp.int32),
            pltpu.VMEM((_CHUNK,), jnp.int32),
            pltpu.VMEM((_CHUNK,), jnp.int32),
            pltpu.VMEM((_CHUNK, d), _F32),
            pltpu.VMEM((_CHUNK, d), _F32),
            pltpu.SemaphoreType.DMA,
            pltpu.SemaphoreType.DMA,
            pltpu.SemaphoreType.DMA,
            pltpu.SemaphoreType.DMA,
        ],
    )
    def agg_kernel(src_hbm, dst_hbm, g_hbm, zeros_hbm, out_hbm,
                   acc_sh, src0, src1, dst0, dst1, rows0, rows1,
                   gsem0, gsem1, ssem0, ssem1):
        c = lax.axis_index("c")
        s = lax.axis_index("s")
        wid = s * nc + c
        base = wid * ept
        src_v, dst_v = (src0, src1), (dst0, dst1)
        rows_v = (rows0, rows1)
        gsem, ssem = (gsem0, gsem1), (ssem0, ssem1)

        for b in range(2):
            off = base + b * _CHUNK
            pltpu.sync_copy(src_hbm.at[pl.ds(off, _CHUNK)], src_v[b])
            pltpu.async_copy(g_hbm.at[src_v[b]], rows_v[b], gsem[b])
            pltpu.sync_copy(dst_hbm.at[pl.ds(off, _CHUNK)], dst_v[b])
        pltpu.sync_copy(zeros_hbm.at[pl.ds(s * rpt, rpt)],
                        acc_sh.at[pl.ds(s * rpt, rpt)])
        plsc.subcore_barrier()

        def body(i, carry):
            for b in range(2):
                off2 = base + (2 * i + b + 2) * _CHUNK
                pltpu.make_async_copy(g_hbm.at[src_v[b]], rows_v[b],
                                      gsem[b]).wait()
                pltpu.async_copy(rows_v[b], acc_sh.at[dst_v[b]], ssem[b],
                                 add=True)

                @pl.when(i < npairs - 1)
                def _(b=b, off2=off2):
                    pltpu.sync_copy(src_hbm.at[pl.ds(off2, _CHUNK)], src_v[b])
                    pltpu.make_async_copy(rows_v[b], acc_sh.at[dst_v[b]],
                                          ssem[b]).wait()
                    pltpu.async_copy(g_hbm.at[src_v[b]], rows_v[b], gsem[b])
                    pltpu.sync_copy(dst_hbm.at[pl.ds(off2, _CHUNK)], dst_v[b])
            return carry

        lax.fori_loop(0, npairs, body, 0)
        for b in range(2):
            pltpu.make_async_copy(rows_v[b], acc_sh.at[dst_v[b]],
                                  ssem[b]).wait()
        plsc.subcore_barrier()
        pltpu.sync_copy(acc_sh.at[pl.ds(s * rpt, rpt)],
                        out_hbm.at[c, pl.ds(s * rpt, rpt)])

    return agg_kernel


def _tc_first_body(degp_ref, x_ref, w_ref, dinv_ref, g_ref):
    deg = degp_ref[0, :, 0:1] + degp_ref[1, :, 0:1] + 1.0
    dinv = lax.rsqrt(deg)
    dinv_ref[...] = dinv
    h = jnp.dot(x_ref[...], w_ref[...],
                preferred_element_type=_F32, precision=lax.Precision.HIGHEST)
    g_ref[...] = dinv * h


def _tc_first(degp, x, w):
    n, d = x.shape
    grid = (n // _BLK,)
    return pl.pallas_call(
        _tc_first_body,
        grid=grid,
        in_specs=[
            pl.BlockSpec((2, _BLK, _DEGW), lambda i: (0, i, 0)),
            pl.BlockSpec((_BLK, d), lambda i: (i, 0)),
            pl.BlockSpec((d, d), lambda i: (0, 0)),
        ],
        out_specs=[
            pl.BlockSpec((_BLK, 1), lambda i: (i, 0)),
            pl.BlockSpec((_BLK, d), lambda i: (i, 0)),
        ],
        out_shape=[
            jax.ShapeDtypeStruct((n, 1), _F32),
            jax.ShapeDtypeStruct((n, d), _F32),
        ],
    )(degp, x, w)


def _tc_mid_body(has_resid, acc_ref, g_ref, dinv_ref, b_ref, w_ref, *rest):
    if has_resid:
        r_ref, xout_ref, gout_ref = rest
    else:
        xout_ref, gout_ref = rest
    dinv = dinv_ref[...]
    a = acc_ref[0] + acc_ref[1] + g_ref[...]
    act = jnp.maximum(dinv * a + b_ref[...], 0.0)
    xout_ref[...] = act
    inp = act + r_ref[...] if has_resid else act
    h = jnp.dot(inp, w_ref[...],
                preferred_element_type=_F32, precision=lax.Precision.HIGHEST)
    gout_ref[...] = dinv * h


def _tc_mid(acc, g, dinv, b, w, resid):
    n, d = g.shape
    grid = (n // _BLK,)
    has_resid = resid is not None
    row = pl.BlockSpec((_BLK, d), lambda i: (i, 0))
    in_specs = [
        pl.BlockSpec((2, _BLK, d), lambda i: (0, i, 0)),
        row,
        pl.BlockSpec((_BLK, 1), lambda i: (i, 0)),
        pl.BlockSpec((1, d), lambda i: (0, 0)),
        pl.BlockSpec((d, d), lambda i: (0, 0)),
    ]
    args = [acc, g, dinv, b.reshape(1, d), w]
    if has_resid:
        in_specs.append(row)
        args.append(resid)
    return pl.pallas_call(
        functools.partial(_tc_mid_body, has_resid),
        grid=grid,
        in_specs=in_specs,
        out_specs=[row, row],
        out_shape=[
            jax.ShapeDtypeStruct((n, d), _F32),
            jax.ShapeDtypeStruct((n, d), _F32),
        ],
    )(*args)


def _tc_last_body(acc_ref, g_ref, dinv_ref, b_ref, out_ref):
    a = acc_ref[0] + acc_ref[1] + g_ref[...]
    out_ref[...] = jnp.maximum(dinv_ref[...] * a + b_ref[...], 0.0)


def _tc_last(acc, g, dinv, b):
    n, d = g.shape
    grid = (n // _BLK,)
    row = pl.BlockSpec((_BLK, d), lambda i: (i, 0))
    return pl.pallas_call(
        _tc_last_body,
        grid=grid,
        in_specs=[
            pl.BlockSpec((2, _BLK, d), lambda i: (0, i, 0)),
            row,
            pl.BlockSpec((_BLK, 1), lambda i: (i, 0)),
            pl.BlockSpec((1, d), lambda i: (0, 0)),
        ],
        out_specs=row,
        out_shape=jax.ShapeDtypeStruct((n, d), _F32),
    )(acc, g, dinv, b.reshape(1, d))


def kernel(x, edge_index, W1, b1, W2, b2, W3, b3, W4, b4):
    n, d = x.shape
    e = edge_index.shape[1]
    nc, ns = _sc_dims()
    nw = nc * ns
    n_pad = -(-(n + 1) // 128) * 128
    ept = -(-e // (nw * 2 * _CHUNK)) * (2 * _CHUNK)
    pad = ept * nw - e
    idt = edge_index.dtype
    src = jnp.concatenate([edge_index[0], jnp.zeros((pad,), idt)])
    dst = jnp.concatenate([edge_index[1], jnp.full((pad,), n, idt)])
    zeros_w = jnp.zeros((n_pad, _DEGW), _F32)
    zeros_d = jnp.zeros((n_pad, d), _F32)

    deg_k = _make_deg_kernel(n_pad, ept)
    agg_k = _make_agg_kernel(n, n_pad, d, ept)

    ones_w = jnp.ones((_CHUNK, _DEGW), _F32)
    degp = deg_k(dst, zeros_w, ones_w)
    dinv, g1 = _tc_first(degp, x, W1)
    acc = agg_k(src, dst, g1, zeros_d)
    x1, g2 = _tc_mid(acc, g1, dinv, b1, W2, None)
    acc = agg_k(src, dst, g2, zeros_d)
    x2, g3 = _tc_mid(acc, g2, dinv, b2, W3, x1)
    acc = agg_k(src, dst, g3, zeros_d)
    x3, g4 = _tc_mid(acc, g3, dinv, b3, W4, x2)
    acc = agg_k(src, dst, g4, zeros_d)
    _x4, g5 = _tc_mid(acc, g4, dinv, b4, W4, x3)
    acc = agg_k(src, dst, g5, zeros_d)
    return _tc_last(acc, g5, dinv, b4)


# speedup vs baseline: 1.0326x; 1.0326x over previous
import functools

import jax
import jax.numpy as jnp
from jax import lax
from jax.experimental import pallas as pl
from jax.experimental.pallas import tpu as pltpu
from jax.experimental.pallas import tpu_sc as plsc

_F32 = jnp.float32
_CHUNK = 128
_DEGW = 128
_BLK = 1000


def _sc_dims():
    info = plsc.get_sparse_core_info()
    return info.num_cores, info.num_subcores


@functools.lru_cache(maxsize=None)
def _make_deg_kernel(n_pad, ept):
    nc, ns = _sc_dims()
    rpt = n_pad // ns
    nchunks = ept // _CHUNK
    mesh = plsc.VectorSubcoreMesh(core_axis_name="c", subcore_axis_name="s")

    nbuf = 4

    @functools.partial(
        pl.kernel,
        out_type=jax.ShapeDtypeStruct((nc, n_pad, _DEGW), _F32),
        mesh=mesh,
        scratch_types=[
            pltpu.VMEM_SHARED((n_pad, _DEGW), _F32),
            pltpu.VMEM((nchunks, _CHUNK), jnp.int32),
            pltpu.VMEM((_CHUNK, _DEGW), _F32),
        ] + [pltpu.SemaphoreType.DMA] * nbuf,
    )
    def deg_kernel(dst_hbm, zeros_hbm, ones_hbm, out_hbm,
                   acc_sh, dst_sl, ones_v, *ssem):
        c = lax.axis_index("c")
        s = lax.axis_index("s")
        wid = s * nc + c
        pltpu.sync_copy(ones_hbm, ones_v)
        pltpu.sync_copy(dst_hbm.at[wid], dst_sl)
        pltpu.sync_copy(zeros_hbm.at[pl.ds(s * rpt, rpt)],
                        acc_sh.at[pl.ds(s * rpt, rpt)])
        plsc.subcore_barrier()

        def body(i, carry):
            for b in range(nbuf):
                j = i * nbuf + b
                pltpu.async_copy(ones_v, acc_sh.at[dst_sl.at[j]], ssem[b],
                                 add=True)

                @pl.when(i > 0)
                def _(b=b, j=j):
                    pltpu.make_async_copy(ones_v, acc_sh.at[dst_sl.at[j]],
                                          ssem[b]).wait()
            return carry

        lax.fori_loop(0, nchunks // nbuf, body, 0)
        for b in range(nbuf):
            pltpu.make_async_copy(ones_v, acc_sh.at[dst_sl.at[0]],
                                  ssem[b]).wait()
        plsc.subcore_barrier()
        pltpu.sync_copy(acc_sh.at[pl.ds(s * rpt, rpt)],
                        out_hbm.at[c, pl.ds(s * rpt, rpt)])

    return deg_kernel


@functools.lru_cache(maxsize=None)
def _make_agg_kernel(n, n_pad, d, ept):
    nc, ns = _sc_dims()
    rpt = n_pad // ns
    nchunks = ept // _CHUNK
    nbuf = 2
    niter = nchunks // nbuf
    mesh = plsc.VectorSubcoreMesh(core_axis_name="c", subcore_axis_name="s")

    @functools.partial(
        pl.kernel,
        out_type=jax.ShapeDtypeStruct((nc, n_pad, d), _F32),
        mesh=mesh,
        scratch_types=[
            pltpu.VMEM_SHARED((n_pad, d), _F32),
            pltpu.VMEM((nchunks, _CHUNK), jnp.int32),
            pltpu.VMEM((nbuf, _CHUNK), jnp.int32),
        ] + [pltpu.VMEM((_CHUNK, d), _F32)] * nbuf
          + [pltpu.SemaphoreType.DMA] * (2 * nbuf),
    )
    def agg_kernel(src_hbm, dst_hbm, g_hbm, zeros_hbm, out_hbm,
                   acc_sh, dst_sl, src_sm, *bufs):
        rows_v = bufs[:nbuf]
        gsem = bufs[nbuf:2 * nbuf]
        ssem = bufs[2 * nbuf:]
        c = lax.axis_index("c")
        s = lax.axis_index("s")
        wid = s * nc + c
        base = wid * ept
        pltpu.sync_copy(dst_hbm.at[wid], dst_sl)
        for b in range(nbuf):
            pltpu.sync_copy(src_hbm.at[pl.ds(base + b * _CHUNK, _CHUNK)],
                            src_sm.at[b])
            pltpu.async_copy(g_hbm.at[src_sm.at[b]], rows_v[b], gsem[b])
        pltpu.sync_copy(zeros_hbm.at[pl.ds(s * rpt, rpt)],
                        acc_sh.at[pl.ds(s * rpt, rpt)])
        plsc.subcore_barrier()

        def body(i, carry):
            for b in range(nbuf):
                j = i * nbuf + b
                pltpu.make_async_copy(g_hbm.at[src_sm.at[b]], rows_v[b],
                                      gsem[b]).wait()
                pltpu.async_copy(rows_v[b], acc_sh.at[dst_sl.at[j]], ssem[b],
                                 add=True)

                @pl.when(i < niter - 1)
                def _(b=b, j=j):
                    pltpu.sync_copy(
                        src_hbm.at[pl.ds(base + (j + nbuf) * _CHUNK, _CHUNK)],
                        src_sm.at[b])
                    pltpu.make_async_copy(rows_v[b], acc_sh.at[dst_sl.at[j]],
                                          ssem[b]).wait()
                    pltpu.async_copy(g_hbm.at[src_sm.at[b]], rows_v[b],
                                     gsem[b])
            return carry

        lax.fori_loop(0, niter, body, 0)
        for b in range(nbuf):
            pltpu.make_async_copy(rows_v[b], acc_sh.at[dst_sl.at[0]],
                                  ssem[b]).wait()
        plsc.subcore_barrier()
        pltpu.sync_copy(acc_sh.at[pl.ds(s * rpt, rpt)],
                        out_hbm.at[c, pl.ds(s * rpt, rpt)])

    return agg_kernel


def _tc_first_body(degp_ref, x_ref, w_ref, dinv_ref, g_ref):
    deg = degp_ref[0, :, 0:1] + degp_ref[1, :, 0:1] + 1.0
    dinv = lax.rsqrt(deg)
    dinv_ref[...] = dinv
    h = jnp.dot(x_ref[...], w_ref[...],
                preferred_element_type=_F32, precision=lax.Precision.HIGHEST)
    g_ref[...] = dinv * h


def _tc_first(degp, x, w):
    n, d = x.shape
    grid = (n // _BLK,)
    return pl.pallas_call(
        _tc_first_body,
        grid=grid,
        in_specs=[
            pl.BlockSpec((2, _BLK, _DEGW), lambda i: (0, i, 0)),
            pl.BlockSpec((_BLK, d), lambda i: (i, 0)),
            pl.BlockSpec((d, d), lambda i: (0, 0)),
        ],
        out_specs=[
            pl.BlockSpec((_BLK, 1), lambda i: (i, 0)),
            pl.BlockSpec((_BLK, d), lambda i: (i, 0)),
        ],
        out_shape=[
            jax.ShapeDtypeStruct((n, 1), _F32),
            jax.ShapeDtypeStruct((n, d), _F32),
        ],
    )(degp, x, w)


def _tc_mid_body(has_resid, acc_ref, g_ref, dinv_ref, b_ref, w_ref, *rest):
    if has_resid:
        r_ref, xout_ref, gout_ref = rest
    else:
        xout_ref, gout_ref = rest
    dinv = dinv_ref[...]
    a = acc_ref[0] + acc_ref[1] + g_ref[...]
    act = jnp.maximum(dinv * a + b_ref[...], 0.0)
    xout_ref[...] = act
    inp = act + r_ref[...] if has_resid else act
    h = jnp.dot(inp, w_ref[...],
                preferred_element_type=_F32, precision=lax.Precision.HIGHEST)
    gout_ref[...] = dinv * h


def _tc_mid(acc, g, dinv, b, w, resid):
    n, d = g.shape
    grid = (n // _BLK,)
    has_resid = resid is not None
    row = pl.BlockSpec((_BLK, d), lambda i: (i, 0))
    in_specs = [
        pl.BlockSpec((2, _BLK, d), lambda i: (0, i, 0)),
        row,
        pl.BlockSpec((_BLK, 1), lambda i: (i, 0)),
        pl.BlockSpec((1, d), lambda i: (0, 0)),
        pl.BlockSpec((d, d), lambda i: (0, 0)),
    ]
    args = [acc, g, dinv, b.reshape(1, d), w]
    if has_resid:
        in_specs.append(row)
        args.append(resid)
    return pl.pallas_call(
        functools.partial(_tc_mid_body, has_resid),
        grid=grid,
        in_specs=in_specs,
        out_specs=[row, row],
        out_shape=[
            jax.ShapeDtypeStruct((n, d), _F32),
            jax.ShapeDtypeStruct((n, d), _F32),
        ],
    )(*args)


def _tc_last_body(acc_ref, g_ref, dinv_ref, b_ref, out_ref):
    a = acc_ref[0] + acc_ref[1] + g_ref[...]
    out_ref[...] = jnp.maximum(dinv_ref[...] * a + b_ref[...], 0.0)


def _tc_last(acc, g, dinv, b):
    n, d = g.shape
    grid = (n // _BLK,)
    row = pl.BlockSpec((_BLK, d), lambda i: (i, 0))
    return pl.pallas_call(
        _tc_last_body,
        grid=grid,
        in_specs=[
            pl.BlockSpec((2, _BLK, d), lambda i: (0, i, 0)),
            row,
            pl.BlockSpec((_BLK, 1), lambda i: (i, 0)),
            pl.BlockSpec((1, d), lambda i: (0, 0)),
        ],
        out_specs=row,
        out_shape=jax.ShapeDtypeStruct((n, d), _F32),
    )(acc, g, dinv, b.reshape(1, d))


def kernel(x, edge_index, W1, b1, W2, b2, W3, b3, W4, b4):
    n, d = x.shape
    e = edge_index.shape[1]
    nc, ns = _sc_dims()
    nw = nc * ns
    n_pad = -(-(n + 1) // 128) * 128
    ept = -(-e // (nw * 4 * _CHUNK)) * (4 * _CHUNK)
    pad = ept * nw - e
    idt = edge_index.dtype
    nch = ept // _CHUNK
    junk = n + jnp.arange(pad, dtype=idt) % (n_pad - n)
    src = jnp.concatenate([edge_index[0], jnp.zeros((pad,), idt)])
    dst = jnp.concatenate([edge_index[1], junk]).reshape(nw, nch, _CHUNK)
    zeros_w = jnp.zeros((n_pad, _DEGW), _F32)
    zeros_d = jnp.zeros((n_pad, d), _F32)

    deg_k = _make_deg_kernel(n_pad, ept)
    agg_k = _make_agg_kernel(n, n_pad, d, ept)

    ones_w = jnp.ones((_CHUNK, _DEGW), _F32)
    degp = deg_k(dst, zeros_w, ones_w)
    dinv, g1 = _tc_first(degp, x, W1)
    acc = agg_k(src, dst, g1, zeros_d)
    x1, g2 = _tc_mid(acc, g1, dinv, b1, W2, None)
    acc = agg_k(src, dst, g2, zeros_d)
    x2, g3 = _tc_mid(acc, g2, dinv, b2, W3, x1)
    acc = agg_k(src, dst, g3, zeros_d)
    x3, g4 = _tc_mid(acc, g3, dinv, b3, W4, x2)
    acc = agg_k(src, dst, g4, zeros_d)
    _x4, g5 = _tc_mid(acc, g4, dinv, b4, W4, x3)
    acc = agg_k(src, dst, g5, zeros_d)
    return _tc_last(acc, g5, dinv, b4)


# speedup vs baseline: 3.8017x; 3.6816x over previous
import functools

import jax
import jax.numpy as jnp
from jax import lax
from jax.experimental import pallas as pl
from jax.experimental.pallas import tpu as pltpu
from jax.experimental.pallas import tpu_sc as plsc

_F32 = jnp.float32
_CHUNK = 88
_DEGW = 128
_BLK = 1000


def _sc_dims():
    info = plsc.get_sparse_core_info()
    return info.num_cores, info.num_subcores


@functools.lru_cache(maxsize=None)
def _make_deg_kernel(n_pad, ept):
    nc, ns = _sc_dims()
    rpt = n_pad // ns
    nchunks = ept // _CHUNK
    mesh = plsc.VectorSubcoreMesh(core_axis_name="c", subcore_axis_name="s")

    nbuf = 3

    @functools.partial(
        pl.kernel,
        out_type=jax.ShapeDtypeStruct((nc, n_pad, _DEGW), _F32),
        mesh=mesh,
        scratch_types=[
            pltpu.VMEM_SHARED((n_pad, _DEGW), _F32),
            pltpu.VMEM((nchunks, _CHUNK), jnp.int32),
            pltpu.VMEM((_CHUNK, _DEGW), _F32),
        ] + [pltpu.SemaphoreType.DMA] * nbuf,
    )
    def deg_kernel(dst_hbm, zeros_hbm, ones_hbm, out_hbm,
                   acc_sh, dst_sl, ones_v, *ssem):
        c = lax.axis_index("c")
        s = lax.axis_index("s")
        wid = s * nc + c
        pltpu.sync_copy(ones_hbm, ones_v)
        pltpu.sync_copy(dst_hbm.at[wid], dst_sl)
        pltpu.sync_copy(zeros_hbm.at[pl.ds(s * rpt, rpt)],
                        acc_sh.at[pl.ds(s * rpt, rpt)])
        plsc.subcore_barrier()

        def body(i, carry):
            for b in range(nbuf):
                j = i * nbuf + b
                pltpu.async_copy(ones_v, acc_sh.at[dst_sl.at[j]], ssem[b],
                                 add=True)

                @pl.when(i > 0)
                def _(b=b, j=j):
                    pltpu.make_async_copy(ones_v, acc_sh.at[dst_sl.at[j]],
                                          ssem[b]).wait()
            return carry

        lax.fori_loop(0, nchunks // nbuf, body, 0)
        for b in range(nbuf):
            pltpu.make_async_copy(ones_v, acc_sh.at[dst_sl.at[0]],
                                  ssem[b]).wait()
        plsc.subcore_barrier()
        pltpu.sync_copy(acc_sh.at[pl.ds(s * rpt, rpt)],
                        out_hbm.at[c, pl.ds(s * rpt, rpt)])

    return deg_kernel


@functools.lru_cache(maxsize=None)
def _make_agg_kernel(n, n_pad, d, ept):
    nc, ns = _sc_dims()
    rpt = n_pad // ns
    nchunks = ept // _CHUNK
    nbuf = 3
    niter = nchunks // nbuf
    gtail = n - (ns - 1) * rpt
    mesh = plsc.VectorSubcoreMesh(core_axis_name="c", subcore_axis_name="s")

    @functools.partial(
        pl.kernel,
        out_type=jax.ShapeDtypeStruct((nc, n_pad, d), _F32),
        mesh=mesh,
        scratch_types=[
            pltpu.VMEM_SHARED((n_pad, d), _F32),
            pltpu.VMEM((nchunks, _CHUNK), jnp.int32),
            pltpu.VMEM((nbuf, _CHUNK), jnp.int32),
        ] + [pltpu.VMEM((_CHUNK, d), _F32)] * nbuf
          + [pltpu.SemaphoreType.DMA] * (2 * nbuf),
    )
    def agg_kernel(src_hbm, dst_hbm, g_hbm, zeros_hbm, out_hbm,
                   acc_sh, dst_sl, src_sm, *bufs):
        rows_v = bufs[:nbuf]
        gsem = bufs[nbuf:2 * nbuf]
        ssem = bufs[2 * nbuf:]
        c = lax.axis_index("c")
        s = lax.axis_index("s")
        wid = s * nc + c
        base = wid * ept
        pltpu.sync_copy(dst_hbm.at[wid], dst_sl)
        for b in range(nbuf):
            pltpu.sync_copy(src_hbm.at[pl.ds(base + b * _CHUNK, _CHUNK)],
                            src_sm.at[b])
            pltpu.async_copy(g_hbm.at[src_sm.at[b]], rows_v[b], gsem[b])
        @pl.when(jnp.logical_and(c == 0, s < ns - 1))
        def _():
            pltpu.sync_copy(g_hbm.at[pl.ds(s * rpt, rpt)],
                            acc_sh.at[pl.ds(s * rpt, rpt)])
        @pl.when(jnp.logical_and(c == 0, s == ns - 1))
        def _():
            pltpu.sync_copy(g_hbm.at[pl.ds(s * rpt, gtail)],
                            acc_sh.at[pl.ds(s * rpt, gtail)])
            pltpu.sync_copy(zeros_hbm.at[pl.ds(s * rpt + gtail, rpt - gtail)],
                            acc_sh.at[pl.ds(s * rpt + gtail, rpt - gtail)])
        @pl.when(c == 1)
        def _():
            pltpu.sync_copy(zeros_hbm.at[pl.ds(s * rpt, rpt)],
                            acc_sh.at[pl.ds(s * rpt, rpt)])
        plsc.subcore_barrier()

        def body(i, carry):
            for b in range(nbuf):
                j = i * nbuf + b
                pltpu.make_async_copy(g_hbm.at[src_sm.at[b]], rows_v[b],
                                      gsem[b]).wait()
                pltpu.async_copy(rows_v[b], acc_sh.at[dst_sl.at[j]], ssem[b],
                                 add=True)

                @pl.when(i < niter - 1)
                def _(b=b, j=j):
                    pltpu.sync_copy(
                        src_hbm.at[pl.ds(base + (j + nbuf) * _CHUNK, _CHUNK)],
                        src_sm.at[b])
                    pltpu.make_async_copy(rows_v[b], acc_sh.at[dst_sl.at[j]],
                                          ssem[b]).wait()
                    pltpu.async_copy(g_hbm.at[src_sm.at[b]], rows_v[b],
                                     gsem[b])
            return carry

        lax.fori_loop(0, niter, body, 0)
        for b in range(nbuf):
            pltpu.make_async_copy(rows_v[b], acc_sh.at[dst_sl.at[0]],
                                  ssem[b]).wait()
        plsc.subcore_barrier()
        pltpu.sync_copy(acc_sh.at[pl.ds(s * rpt, rpt)],
                        out_hbm.at[c, pl.ds(s * rpt, rpt)])

    return agg_kernel


def _tc_first_body(degp_ref, x_ref, w_ref, dinv_ref, g_ref):
    deg = degp_ref[0, :, 0:1] + degp_ref[1, :, 0:1] + 1.0
    dinv = lax.rsqrt(deg)
    dinv_ref[...] = dinv
    h = jnp.dot(x_ref[...], w_ref[...],
                preferred_element_type=_F32, precision=lax.Precision.HIGHEST)
    g_ref[...] = dinv * h


def _tc_first(degp, x, w):
    n, d = x.shape
    grid = (n // _BLK,)
    return pl.pallas_call(
        _tc_first_body,
        grid=grid,
        in_specs=[
            pl.BlockSpec((2, _BLK, _DEGW), lambda i: (0, i, 0)),
            pl.BlockSpec((_BLK, d), lambda i: (i, 0)),
            pl.BlockSpec((d, d), lambda i: (0, 0)),
        ],
        out_specs=[
            pl.BlockSpec((_BLK, 1), lambda i: (i, 0)),
            pl.BlockSpec((_BLK, d), lambda i: (i, 0)),
        ],
        out_shape=[
            jax.ShapeDtypeStruct((n, 1), _F32),
            jax.ShapeDtypeStruct((n, d), _F32),
        ],
    )(degp, x, w)


def _tc_mid_body(has_resid, write_x, acc_ref, dinv_ref, b_ref, w_ref, *rest):
    if has_resid:
        r_ref, rest = rest[0], rest[1:]
    if write_x:
        xout_ref, gout_ref = rest
    else:
        gout_ref, = rest
    dinv = dinv_ref[...]
    a = acc_ref[0] + acc_ref[1]
    act = jnp.maximum(dinv * a + b_ref[...], 0.0)
    if write_x:
        xout_ref[...] = act
    inp = act + r_ref[...] if has_resid else act
    h = jnp.dot(inp, w_ref[...],
                preferred_element_type=_F32, precision=lax.Precision.HIGHEST)
    gout_ref[...] = dinv * h


def _tc_mid(acc, dinv, b, w, resid, write_x=True):
    n, d = dinv.shape[0], acc.shape[2]
    grid = (n // _BLK,)
    has_resid = resid is not None
    row = pl.BlockSpec((_BLK, d), lambda i: (i, 0))
    in_specs = [
        pl.BlockSpec((2, _BLK, d), lambda i: (0, i, 0)),
        pl.BlockSpec((_BLK, 1), lambda i: (i, 0)),
        pl.BlockSpec((1, d), lambda i: (0, 0)),
        pl.BlockSpec((d, d), lambda i: (0, 0)),
    ]
    args = [acc, dinv, b.reshape(1, d), w]
    if has_resid:
        in_specs.append(row)
        args.append(resid)
    n_out = 2 if write_x else 1
    return pl.pallas_call(
        functools.partial(_tc_mid_body, has_resid, write_x),
        grid=grid,
        in_specs=in_specs,
        out_specs=[row] * n_out,
        out_shape=[jax.ShapeDtypeStruct((n, d), _F32)] * n_out,
    )(*args)


def _tc_last_body(acc_ref, dinv_ref, b_ref, out_ref):
    a = acc_ref[0] + acc_ref[1]
    out_ref[...] = jnp.maximum(dinv_ref[...] * a + b_ref[...], 0.0)


def _tc_last(acc, dinv, b):
    n, d = dinv.shape[0], acc.shape[2]
    grid = (n // _BLK,)
    row = pl.BlockSpec((_BLK, d), lambda i: (i, 0))
    return pl.pallas_call(
        _tc_last_body,
        grid=grid,
        in_specs=[
            pl.BlockSpec((2, _BLK, d), lambda i: (0, i, 0)),
            pl.BlockSpec((_BLK, 1), lambda i: (i, 0)),
            pl.BlockSpec((1, d), lambda i: (0, 0)),
        ],
        out_specs=row,
        out_shape=jax.ShapeDtypeStruct((n, d), _F32),
    )(acc, dinv, b.reshape(1, d))


def kernel(x, edge_index, W1, b1, W2, b2, W3, b3, W4, b4):
    n, d = x.shape
    e = edge_index.shape[1]
    nc, ns = _sc_dims()
    nw = nc * ns
    n_pad = -(-(n + 1) // 128) * 128
    ept = -(-e // (nw * 3 * _CHUNK)) * (3 * _CHUNK)
    pad = ept * nw - e
    idt = edge_index.dtype
    nch = ept // _CHUNK
    import numpy as _np
    junk = jnp.asarray(n + _np.arange(pad) % (n_pad - n), idt)
    src_pad = jnp.asarray(_np.arange(pad) % n, idt)
    src = jnp.concatenate([edge_index[0], src_pad])
    dst3 = jnp.concatenate([edge_index[1], junk]).reshape(nw, nch, _CHUNK)
    zeros_w = jnp.zeros((n_pad, _DEGW), _F32)
    zeros_d = jnp.zeros((n_pad, d), _F32)

    deg_k = _make_deg_kernel(n_pad, ept)
    agg_k = _make_agg_kernel(n, n_pad, d, ept)

    ones_w = jnp.ones((_CHUNK, _DEGW), _F32)
    degp = deg_k(dst3, zeros_w, ones_w)
    dinv, g1 = _tc_first(degp, x, W1)
    acc = agg_k(src, dst3, g1, zeros_d)
    x1, g2 = _tc_mid(acc, dinv, b1, W2, None)
    acc = agg_k(src, dst3, g2, zeros_d)
    x2, g3 = _tc_mid(acc, dinv, b2, W3, x1)
    acc = agg_k(src, dst3, g3, zeros_d)
    x3, g4 = _tc_mid(acc, dinv, b3, W4, x2)
    acc = agg_k(src, dst3, g4, zeros_d)
    (g5,) = _tc_mid(acc, dinv, b4, W4, x3, write_x=False)
    acc = agg_k(src, dst3, g5, zeros_d)
    return _tc_last(acc, dinv, b4)
